# baseline (speedup 1.0000x reference)
import functools

import jax
import jax.numpy as jnp
from jax import lax
from jax.experimental import pallas as pl
from jax.experimental.pallas import tpu as pltpu
from jax.experimental.pallas import tpu_sc as plsc

N = 10000
D = 128
E = 320000
NC = 2
NS = 16
NW = NC * NS
CH = 64
NCHUNK = 162
EPAD = NW * NCHUNK * CH
NPAD = 10496
ROWS_PER_TILE = NPAD // NS
DUMMY = N
MB = 656
GRID = NPAD // MB


def _sc_mesh():
    return plsc.VectorSubcoreMesh(core_axis_name="c", subcore_axis_name="s",
                                  num_cores=NC, num_subcores=NS)


_SC_PARAMS = pltpu.CompilerParams(use_tc_tiling_on_sc=False)


def _pre_body(row_hbm, col_hbm, zeros16_hbm, ceff_hbm, deg2_hbm,
              rowv, colv, ceffv, reffv, onesv, degsh):
    c = lax.axis_index("c")
    s = lax.axis_index("s")
    tid = c * NS + s
    pltpu.sync_copy(row_hbm.at[tid], rowv)
    pltpu.sync_copy(col_hbm.at[tid], colv)
    pltpu.sync_copy(zeros16_hbm.at[pl.ds(s * ROWS_PER_TILE, ROWS_PER_TILE)],
                    degsh.at[pl.ds(s * ROWS_PER_TILE, ROWS_PER_TILE)])
    lanes = lax.iota(jnp.int32, 16)

    sub = CH // 16

    def cb(i, carry):
        j = i // sub
        k = (i % sub) * 16
        r = rowv[j, pl.ds(k, 16)]
        cc = colv[j, pl.ds(k, 16)]
        m = r != cc
        dv = DUMMY + lax.rem(i * 16 + lanes, NPAD - N)
        ceffv[j, pl.ds(k, 16)] = jnp.where(m, cc, dv)
        reffv[j, pl.ds(k, 16)] = jnp.where(m, r, dv)
        return carry

    lax.fori_loop(0, NCHUNK * sub, cb, 0)

    def ob(i, carry):
        onesv[i] = jnp.where(lanes == 0, 1.0, 0.0).astype(jnp.float32)
        return carry

    lax.fori_loop(0, CH, ob, 0)
    plsc.subcore_barrier()

    def sb(j, carry):
        pltpu.sync_copy(onesv, degsh.at[reffv.at[j]], add=True)
        return carry

    lax.fori_loop(0, NCHUNK, sb, 0)
    plsc.subcore_barrier()
    base = c * NPAD + s * ROWS_PER_TILE
    pltpu.sync_copy(degsh.at[pl.ds(s * ROWS_PER_TILE, ROWS_PER_TILE)],
                    deg2_hbm.at[pl.ds(base, ROWS_PER_TILE)])
    pltpu.sync_copy(ceffv, ceff_hbm.at[tid])


def _preprocess(row_p, col_p, zeros16):
    return pl.kernel(
        _pre_body,
        out_type=(
            jax.ShapeDtypeStruct((NW, NCHUNK, CH), jnp.int32),
            jax.ShapeDtypeStruct((NC * NPAD, 16), jnp.float32),
        ),
        mesh=_sc_mesh(),
        scratch_types=[
            pltpu.VMEM((NCHUNK, CH), jnp.int32),
            pltpu.VMEM((NCHUNK, CH), jnp.int32),
            pltpu.VMEM((NCHUNK, CH), jnp.int32),
            pltpu.VMEM((NCHUNK, CH), jnp.int32),
            pltpu.VMEM((CH, 16), jnp.float32),
            pltpu.VMEM_SHARED((NPAD, 16), jnp.float32),
        ],
        compiler_params=_SC_PARAMS,
    )(row_p, col_p, zeros16)


def _agg_body(s_hbm, row_hbm, ceff_hbm, zerosd_hbm, out_hbm,
              rowv, ceffv, buf0, buf1, buf2,
              acc, sem0, sem1, sem2):
    c = lax.axis_index("c")
    s = lax.axis_index("s")
    tid = c * NS + s
    bufs = (buf0, buf1, buf2)
    sems = (sem0, sem1, sem2)
    pltpu.sync_copy(row_hbm.at[tid], rowv)
    pltpu.sync_copy(ceff_hbm.at[tid], ceffv)

    def issue(j, k):
        pltpu.async_copy(s_hbm.at[rowv.at[j]], bufs[k], sems[k])

    def wait(k):
        pltpu.make_async_copy(s_hbm.at[pl.ds(0, CH)], bufs[k], sems[k]).wait()

    def scat(j, k):
        pltpu.sync_copy(bufs[k], acc.at[ceffv.at[j]], add=True)

    pltpu.sync_copy(zerosd_hbm.at[pl.ds(s * ROWS_PER_TILE, ROWS_PER_TILE)],
                    acc.at[pl.ds(s * ROWS_PER_TILE, ROWS_PER_TILE)])
    plsc.subcore_barrier()
    issue(0, 0)
    issue(1, 1)
    issue(2, 2)
    ntri = NCHUNK // 3

    def tri(i, carry):
        for k in range(3):
            wait(k)
            scat(3 * i + k, k)
            issue(3 * i + k + 3, k)
        return carry

    lax.fori_loop(0, ntri - 1, tri, 0)
    for k in range(3):
        wait(k)
        scat(3 * (ntri - 1) + k, k)
    plsc.subcore_barrier()
    base = c * NPAD + s * ROWS_PER_TILE
    pltpu.sync_copy(acc.at[pl.ds(s * ROWS_PER_TILE, ROWS_PER_TILE)],
                    out_hbm.at[pl.ds(base, ROWS_PER_TILE)])


def _aggregate(s_nodes, row_p, ceff, zerosd):
    return pl.kernel(
        _agg_body,
        out_type=jax.ShapeDtypeStruct((NC * NPAD, D), jnp.float32),
        mesh=_sc_mesh(),
        scratch_types=[
            pltpu.VMEM((NCHUNK, CH), jnp.int32),
            pltpu.VMEM((NCHUNK, CH), jnp.int32),
            pltpu.VMEM((CH, D), jnp.float32),
            pltpu.VMEM((CH, D), jnp.float32),
            pltpu.VMEM((CH, D), jnp.float32),
            pltpu.VMEM_SHARED((NPAD, D), jnp.float32),
            pltpu.SemaphoreType.DMA,
            pltpu.SemaphoreType.DMA,
            pltpu.SemaphoreType.DMA,
        ],
        compiler_params=_SC_PARAMS,
    )(s_nodes, row_p, ceff, zerosd)


def _fin_body(deg2_ref, dis_ref):
    full = deg2_ref[...]
    d = full[0:NPAD, 0:8] + full[NPAD:2 * NPAD, 0:8]
    r = lax.rsqrt(d)
    row = lax.broadcasted_iota(jnp.int32, (NPAD, 8), 0)
    dis_ref[...] = jnp.where(row < N, r, 0.0)


def _finalize_deg(deg2):
    return pl.pallas_call(
        _fin_body,
        out_shape=jax.ShapeDtypeStruct((NPAD, 8), jnp.float32),
    )(deg2)


def _dot(h, w):
    return lax.dot_general(h, w, (((1,), (1,)), ((), ())),
                           precision=lax.Precision.HIGHEST,
                           preferred_element_type=jnp.float32)


def _lin_body(dis_ref, h_ref, w_ref, b_ref, s_ref):
    dis = dis_ref[...][:, 0:1]
    s_ref[...] = dis * (_dot(h_ref[...], w_ref[...]) + b_ref[...])


def _linear(dis, h, w, b):
    return pl.pallas_call(
        _lin_body,
        grid=(GRID,),
        in_specs=[
            pl.BlockSpec((MB, 8), lambda i: (i, 0)),
            pl.BlockSpec((MB, D), lambda i: (i, 0)),
            pl.BlockSpec((D, D), lambda i: (0, 0)),
            pl.BlockSpec((1, D), lambda i: (0, 0)),
        ],
        out_specs=pl.BlockSpec((MB, D), lambda i: (i, 0)),
        out_shape=jax.ShapeDtypeStruct((NPAD, D), jnp.float32),
    )(dis, h, w, b)


def _resid_lin_body(h_ref, a0_ref, a1_ref, dis_ref, w_ref, b_ref, hn_ref, s_ref):
    dis = dis_ref[...][:, 0:1]
    acc = a0_ref[...] + a1_ref[...]
    hn = h_ref[...] + jnp.maximum(dis * acc, 0.0)
    hn_ref[...] = hn
    s_ref[...] = dis * (_dot(hn, w_ref[...]) + b_ref[...])


def _resid_linear(h, acc2, dis, w, b):
    return pl.pallas_call(
        _resid_lin_body,
        grid=(GRID,),
        in_specs=[
            pl.BlockSpec((MB, D), lambda i: (i, 0)),
            pl.BlockSpec((MB, D), lambda i: (i, 0)),
            pl.BlockSpec((MB, D), lambda i: (i + GRID, 0)),
            pl.BlockSpec((MB, 8), lambda i: (i, 0)),
            pl.BlockSpec((D, D), lambda i: (0, 0)),
            pl.BlockSpec((1, D), lambda i: (0, 0)),
        ],
        out_specs=[
            pl.BlockSpec((MB, D), lambda i: (i, 0)),
            pl.BlockSpec((MB, D), lambda i: (i, 0)),
        ],
        out_shape=[
            jax.ShapeDtypeStruct((NPAD, D), jnp.float32),
            jax.ShapeDtypeStruct((NPAD, D), jnp.float32),
        ],
    )(h, acc2, acc2, dis, w, b)


def _resid_body(h_ref, a0_ref, a1_ref, dis_ref, hn_ref):
    dis = dis_ref[...][:, 0:1]
    acc = a0_ref[...] + a1_ref[...]
    hn_ref[...] = h_ref[...] + jnp.maximum(dis * acc, 0.0)


def _resid(h, acc2, dis):
    return pl.pallas_call(
        _resid_body,
        grid=(GRID,),
        in_specs=[
            pl.BlockSpec((MB, D), lambda i: (i, 0)),
            pl.BlockSpec((MB, D), lambda i: (i, 0)),
            pl.BlockSpec((MB, D), lambda i: (i + GRID, 0)),
            pl.BlockSpec((MB, 8), lambda i: (i, 0)),
        ],
        out_specs=pl.BlockSpec((MB, D), lambda i: (i, 0)),
        out_shape=jax.ShapeDtypeStruct((NPAD, D), jnp.float32),
    )(h, acc2, acc2, dis)


def kernel(x, edge_index, W0, b0, W1, b1, W2, b2):
    x_p = jnp.pad(x, ((0, NPAD - N), (0, 0)))

    pad = jnp.zeros((EPAD - E,), jnp.int32)
    row_p = jnp.concatenate([edge_index[0], pad]).reshape(NW, NCHUNK, CH)
    col_p = jnp.concatenate([edge_index[1], pad]).reshape(NW, NCHUNK, CH)
    zeros16 = jnp.zeros((NPAD, 16), jnp.float32)
    zerosd = jnp.zeros((NPAD, D), jnp.float32)

    ceff, deg2 = _preprocess(row_p, col_p, zeros16)
    dis = _finalize_deg(deg2)

    b0r = b0.reshape(1, D)
    b1r = b1.reshape(1, D)
    b2r = b2.reshape(1, D)

    s0 = _linear(dis, x_p, W0, b0r)
    acc = _aggregate(s0, row_p, ceff, zerosd)
    h1, s1 = _resid_linear(x_p, acc, dis, W1, b1r)
    acc = _aggregate(s1, row_p, ceff, zerosd)
    h2, s2 = _resid_linear(h1, acc, dis, W2, b2r)
    acc = _aggregate(s2, row_p, ceff, zerosd)
    h3 = _resid(h2, acc, dis)
    return h3[:N]


# speedup vs baseline: 2.9880x; 2.9880x over previous
import jax
import jax.numpy as jnp
from jax import lax
from jax.experimental import pallas as pl
from jax.experimental.pallas import tpu as pltpu
from jax.experimental.pallas import tpu_sc as plsc

N = 10000
D = 128
HD = D // 2
E = 320000
NC = 2
NS = 16
NW = NC * NS
CH = 64
NCHUNK = 158
NCH2 = 2 * NCHUNK
EPAD = NW * NCHUNK * CH
NPAD = 10112
ROWS_PER_TILE = NPAD // NS
DUMMY = N
MB = 632
GRID = NPAD // MB


def _sc_mesh():
    return plsc.VectorSubcoreMesh(core_axis_name="c", subcore_axis_name="s",
                                  num_cores=NC, num_subcores=NS)


_SC_PARAMS = pltpu.CompilerParams(use_tc_tiling_on_sc=False)


def _pre_body(row_hbm, col_hbm, zeros16_hbm, ceff_hbm, deg2_hbm,
              rowv, colv, ceffv, reffv, onesv, degsh):
    c = lax.axis_index("c")
    s = lax.axis_index("s")
    tid = c * NS + s
    pltpu.sync_copy(row_hbm.at[tid], rowv)
    pltpu.sync_copy(col_hbm.at[tid], colv)
    pltpu.sync_copy(zeros16_hbm.at[pl.ds(s * ROWS_PER_TILE, ROWS_PER_TILE)],
                    degsh.at[pl.ds(s * ROWS_PER_TILE, ROWS_PER_TILE)])
    lanes = lax.iota(jnp.int32, 16)

    sub = CH // 16

    def cb(i, carry):
        j = i // sub
        k = (i % sub) * 16
        r = rowv[j, pl.ds(k, 16)]
        cc = colv[j, pl.ds(k, 16)]
        m = r != cc
        dv = DUMMY + lax.rem(i * 16 + lanes, NPAD - N)
        ceffv[j, pl.ds(k, 16)] = jnp.where(m, cc, dv)
        reffv[j, pl.ds(k, 16)] = jnp.where(m, r, dv)
        return carry

    lax.fori_loop(0, NCHUNK * sub, cb, 0)

    def ob(i, carry):
        onesv[i] = jnp.where(lanes == 0, 1.0, 0.0).astype(jnp.float32)
        return carry

    lax.fori_loop(0, CH, ob, 0)
    plsc.subcore_barrier()

    def sb(j, carry):
        pltpu.sync_copy(onesv, degsh.at[reffv.at[j]], add=True)
        return carry

    lax.fori_loop(0, NCHUNK, sb, 0)
    plsc.subcore_barrier()
    base = c * NPAD + s * ROWS_PER_TILE
    pltpu.sync_copy(degsh.at[pl.ds(s * ROWS_PER_TILE, ROWS_PER_TILE)],
                    deg2_hbm.at[pl.ds(base, ROWS_PER_TILE)])
    pltpu.sync_copy(ceffv, ceff_hbm.at[tid])


def _preprocess(row_p, col_p, zeros16):
    return pl.kernel(
        _pre_body,
        out_type=(
            jax.ShapeDtypeStruct((NW, NCHUNK, CH), jnp.int32),
            jax.ShapeDtypeStruct((NC * NPAD, 16), jnp.float32),
        ),
        mesh=_sc_mesh(),
        scratch_types=[
            pltpu.VMEM((NCHUNK, CH), jnp.int32),
            pltpu.VMEM((NCHUNK, CH), jnp.int32),
            pltpu.VMEM((NCHUNK, CH), jnp.int32),
            pltpu.VMEM((NCHUNK, CH), jnp.int32),
            pltpu.VMEM((CH, 16), jnp.float32),
            pltpu.VMEM_SHARED((NPAD, 16), jnp.float32),
        ],
        compiler_params=_SC_PARAMS,
    )(row_p, col_p, zeros16)


def _agg_body(s2_hbm, row_hbm, ceff_hbm, zerosd_hbm, out_hbm,
              rowv, ceffv, buf0, buf1,
              table, acc, sem0, sem1):
    c = lax.axis_index("c")
    s = lax.axis_index("s")
    pltpu.sync_copy(row_hbm.at[s], rowv)
    pltpu.sync_copy(ceff_hbm.at[s], ceffv)
    pltpu.sync_copy(s2_hbm.at[pl.ds(c * NPAD + s * ROWS_PER_TILE, ROWS_PER_TILE)],
                    table.at[pl.ds(s * ROWS_PER_TILE, ROWS_PER_TILE)])
    pltpu.sync_copy(zerosd_hbm.at[pl.ds(s * ROWS_PER_TILE, ROWS_PER_TILE)],
                    acc.at[pl.ds(s * ROWS_PER_TILE, ROWS_PER_TILE)])
    plsc.subcore_barrier()

    def issue(j, buf, sem):
        pltpu.async_copy(table.at[rowv.at[j]], buf, sem)

    def wait(buf, sem):
        pltpu.make_async_copy(table.at[pl.ds(0, CH)], buf, sem).wait()

    def scat(j, buf):
        pltpu.sync_copy(buf, acc.at[ceffv.at[j]], add=True)

    issue(0, buf0, sem0)
    issue(1, buf1, sem1)

    def pair(i, carry):
        wait(buf0, sem0)
        scat(2 * i, buf0)
        issue(2 * i + 2, buf0, sem0)
        wait(buf1, sem1)
        scat(2 * i + 1, buf1)
        issue(2 * i + 3, buf1, sem1)
        return carry

    lax.fori_loop(0, NCH2 // 2 - 1, pair, 0)
    wait(buf0, sem0)
    scat(NCH2 - 2, buf0)
    wait(buf1, sem1)
    scat(NCH2 - 1, buf1)
    plsc.subcore_barrier()
    pltpu.sync_copy(acc.at[pl.ds(s * ROWS_PER_TILE, ROWS_PER_TILE)],
                    out_hbm.at[pl.ds(c * NPAD + s * ROWS_PER_TILE, ROWS_PER_TILE)])


def _aggregate(s2, row_a, ceff_a, zerosd):
    return pl.kernel(
        _agg_body,
        out_type=jax.ShapeDtypeStruct((NC * NPAD, HD), jnp.float32),
        mesh=_sc_mesh(),
        scratch_types=[
            pltpu.VMEM((NCH2, CH), jnp.int32),
            pltpu.VMEM((NCH2, CH), jnp.int32),
            pltpu.VMEM((CH, HD), jnp.float32),
            pltpu.VMEM((CH, HD), jnp.float32),
            pltpu.VMEM_SHARED((NPAD, HD), jnp.float32),
            pltpu.VMEM_SHARED((NPAD, HD), jnp.float32),
            pltpu.SemaphoreType.DMA,
            pltpu.SemaphoreType.DMA,
        ],
        compiler_params=_SC_PARAMS,
    )(s2, row_a, ceff_a, zerosd)


def _fin_body(deg2_ref, dis_ref):
    full = deg2_ref[...]
    d = full[0:NPAD, 0:8] + full[NPAD:2 * NPAD, 0:8]
    r = lax.rsqrt(d)
    row = lax.broadcasted_iota(jnp.int32, (NPAD, 8), 0)
    dis_ref[...] = jnp.where(row < N, r, 0.0)


def _finalize_deg(deg2):
    return pl.pallas_call(
        _fin_body,
        out_shape=jax.ShapeDtypeStruct((NPAD, 8), jnp.float32),
    )(deg2)


def _dot(h, w):
    return lax.dot_general(h, w, (((1,), (1,)), ((), ())),
                           precision=lax.Precision.HIGHEST,
                           preferred_element_type=jnp.float32)


def _lin_body(dis_ref, h_ref, w_ref, b_ref, s_ref):
    dis = dis_ref[...][:, 0:1]
    v = dis * (_dot(h_ref[...], w_ref[...]) + b_ref[...])
    s_ref[0, :, :] = v[:, 0:HD]
    s_ref[1, :, :] = v[:, HD:D]


def _linear(dis, h, w, b):
    return pl.pallas_call(
        _lin_body,
        grid=(GRID,),
        in_specs=[
            pl.BlockSpec((MB, 8), lambda i: (i, 0)),
            pl.BlockSpec((MB, D), lambda i: (i, 0)),
            pl.BlockSpec((D, D), lambda i: (0, 0)),
            pl.BlockSpec((1, D), lambda i: (0, 0)),
        ],
        out_specs=pl.BlockSpec((NC, MB, HD), lambda i: (0, i, 0)),
        out_shape=jax.ShapeDtypeStruct((NC, NPAD, HD), jnp.float32),
    )(dis, h, w, b)


def _resid_lin_body(h_ref, a0_ref, a1_ref, dis_ref, w_ref, b_ref, hn_ref, s_ref):
    dis = dis_ref[...][:, 0:1]
    acc = jnp.concatenate([a0_ref[...], a1_ref[...]], axis=1)
    hn = h_ref[...] + jnp.maximum(dis * acc, 0.0)
    hn_ref[...] = hn
    v = dis * (_dot(hn, w_ref[...]) + b_ref[...])
    s_ref[0, :, :] = v[:, 0:HD]
    s_ref[1, :, :] = v[:, HD:D]


def _resid_linear(h, acc2, dis, w, b):
    return pl.pallas_call(
        _resid_lin_body,
        grid=(GRID,),
        in_specs=[
            pl.BlockSpec((MB, D), lambda i: (i, 0)),
            pl.BlockSpec((MB, HD), lambda i: (i, 0)),
            pl.BlockSpec((MB, HD), lambda i: (i + GRID, 0)),
            pl.BlockSpec((MB, 8), lambda i: (i, 0)),
            pl.BlockSpec((D, D), lambda i: (0, 0)),
            pl.BlockSpec((1, D), lambda i: (0, 0)),
        ],
        out_specs=[
            pl.BlockSpec((MB, D), lambda i: (i, 0)),
            pl.BlockSpec((NC, MB, HD), lambda i: (0, i, 0)),
        ],
        out_shape=[
            jax.ShapeDtypeStruct((NPAD, D), jnp.float32),
            jax.ShapeDtypeStruct((NC, NPAD, HD), jnp.float32),
        ],
    )(h, acc2, acc2, dis, w, b)


def _resid_body(h_ref, a0_ref, a1_ref, dis_ref, hn_ref):
    dis = dis_ref[...][:, 0:1]
    acc = jnp.concatenate([a0_ref[...], a1_ref[...]], axis=1)
    hn_ref[...] = h_ref[...] + jnp.maximum(dis * acc, 0.0)


def _resid(h, acc2, dis):
    return pl.pallas_call(
        _resid_body,
        grid=(GRID,),
        in_specs=[
            pl.BlockSpec((MB, D), lambda i: (i, 0)),
            pl.BlockSpec((MB, HD), lambda i: (i, 0)),
            pl.BlockSpec((MB, HD), lambda i: (i + GRID, 0)),
            pl.BlockSpec((MB, 8), lambda i: (i, 0)),
        ],
        out_specs=pl.BlockSpec((MB, D), lambda i: (i, 0)),
        out_shape=jax.ShapeDtypeStruct((NPAD, D), jnp.float32),
    )(h, acc2, acc2, dis)


def kernel(x, edge_index, W0, b0, W1, b1, W2, b2):
    x_p = jnp.pad(x, ((0, NPAD - N), (0, 0)))
    pad = jnp.zeros((EPAD - E,), jnp.int32)
    row_p = jnp.concatenate([edge_index[0], pad]).reshape(NW, NCHUNK, CH)
    col_p = jnp.concatenate([edge_index[1], pad]).reshape(NW, NCHUNK, CH)
    zeros16 = jnp.zeros((NPAD, 16), jnp.float32)
    zerosd = jnp.zeros((NPAD, HD), jnp.float32)

    ceff, deg2 = _preprocess(row_p, col_p, zeros16)
    dis = _finalize_deg(deg2)

    row_a = row_p.reshape(NS, NCH2, CH)
    ceff_a = ceff.reshape(NS, NCH2, CH)

    b0r = b0.reshape(1, D)
    b1r = b1.reshape(1, D)
    b2r = b2.reshape(1, D)

    s3 = _linear(dis, x_p, W0, b0r).reshape(NC * NPAD, HD)
    acc = _aggregate(s3, row_a, ceff_a, zerosd)
    h1, s3 = _resid_linear(x_p, acc, dis, W1, b1r)
    acc = _aggregate(s3.reshape(NC * NPAD, HD), row_a, ceff_a, zerosd)
    h2, s3 = _resid_linear(h1, acc, dis, W2, b2r)
    acc = _aggregate(s3.reshape(NC * NPAD, HD), row_a, ceff_a, zerosd)
    h3 = _resid(h2, acc, dis)
    return h3[:N]


# speedup vs baseline: 3.3092x; 1.1075x over previous
import jax
import jax.numpy as jnp
from jax import lax
from jax.experimental import pallas as pl
from jax.experimental.pallas import tpu as pltpu
from jax.experimental.pallas import tpu_sc as plsc

N = 10000
D = 128
HD = D // 2
E = 320000
NC = 2
NS = 16
NW = NC * NS
CH = 64
NCHUNK = 158
NCH2 = 2 * NCHUNK
EPAD = NW * NCHUNK * CH
NPAD = 10112
ROWS_PER_TILE = NPAD // NS
DUMMY = N
MB = 632
GRID = NPAD // MB


def _sc_mesh():
    return plsc.VectorSubcoreMesh(core_axis_name="c", subcore_axis_name="s",
                                  num_cores=NC, num_subcores=NS)


_SC_PARAMS = pltpu.CompilerParams(use_tc_tiling_on_sc=False)


def _pre_body(row_hbm, col_hbm, zeros16_hbm, ceff_hbm, deg2_hbm,
              rowv, colv, ceffv, reffv, onesv, degsh):
    c = lax.axis_index("c")
    s = lax.axis_index("s")
    tid = c * NS + s
    pltpu.sync_copy(row_hbm.at[tid], rowv)
    pltpu.sync_copy(col_hbm.at[tid], colv)
    pltpu.sync_copy(zeros16_hbm.at[pl.ds(s * ROWS_PER_TILE, ROWS_PER_TILE)],
                    degsh.at[pl.ds(s * ROWS_PER_TILE, ROWS_PER_TILE)])
    lanes = lax.iota(jnp.int32, 16)

    sub = CH // 16

    def cb(i, carry):
        j = i // sub
        k = (i % sub) * 16
        r = rowv[j, pl.ds(k, 16)]
        cc = colv[j, pl.ds(k, 16)]
        m = r != cc
        dv = DUMMY + lax.rem(i * 16 + lanes, NPAD - N)
        ceffv[j, pl.ds(k, 16)] = jnp.where(m, cc, dv)
        reffv[j, pl.ds(k, 16)] = jnp.where(m, r, dv)
        return carry

    lax.fori_loop(0, NCHUNK * sub, cb, 0)

    def ob(i, carry):
        onesv[i] = jnp.where(lanes == 0, 1.0, 0.0).astype(jnp.float32)
        return carry

    lax.fori_loop(0, CH, ob, 0)
    plsc.subcore_barrier()

    def sb(j, carry):
        pltpu.sync_copy(onesv, degsh.at[reffv.at[j]], add=True)
        return carry

    lax.fori_loop(0, NCHUNK, sb, 0)
    plsc.subcore_barrier()
    base = c * NPAD + s * ROWS_PER_TILE
    pltpu.sync_copy(degsh.at[pl.ds(s * ROWS_PER_TILE, ROWS_PER_TILE)],
                    deg2_hbm.at[pl.ds(base, ROWS_PER_TILE)])
    pltpu.sync_copy(ceffv, ceff_hbm.at[tid])


def _preprocess(row_p, col_p, zeros16):
    return pl.kernel(
        _pre_body,
        out_type=(
            jax.ShapeDtypeStruct((NW, NCHUNK, CH), jnp.int32),
            jax.ShapeDtypeStruct((NC * NPAD, 16), jnp.float32),
        ),
        mesh=_sc_mesh(),
        scratch_types=[
            pltpu.VMEM((NCHUNK, CH), jnp.int32),
            pltpu.VMEM((NCHUNK, CH), jnp.int32),
            pltpu.VMEM((NCHUNK, CH), jnp.int32),
            pltpu.VMEM((NCHUNK, CH), jnp.int32),
            pltpu.VMEM((CH, 16), jnp.float32),
            pltpu.VMEM_SHARED((NPAD, 16), jnp.float32),
        ],
        compiler_params=_SC_PARAMS,
    )(row_p, col_p, zeros16)


def _agg_body(s2_hbm, row_hbm, ceff_hbm, zerosd_hbm, out_hbm,
              rowv, ceffv, buf0, buf1,
              table, acc, sem0, sem1):
    c = lax.axis_index("c")
    s = lax.axis_index("s")
    pltpu.sync_copy(row_hbm.at[s], rowv)
    pltpu.sync_copy(ceff_hbm.at[s], ceffv)
    pltpu.sync_copy(s2_hbm.at[pl.ds(s * ROWS_PER_TILE, ROWS_PER_TILE),
                              pl.ds(c * HD, HD)],
                    table.at[pl.ds(s * ROWS_PER_TILE, ROWS_PER_TILE)])
    pltpu.sync_copy(zerosd_hbm.at[pl.ds(s * ROWS_PER_TILE, ROWS_PER_TILE)],
                    acc.at[pl.ds(s * ROWS_PER_TILE, ROWS_PER_TILE)])
    plsc.subcore_barrier()

    def issue(j, buf, sem):
        pltpu.async_copy(table.at[rowv.at[j]], buf, sem)

    def wait(buf, sem):
        pltpu.make_async_copy(table.at[pl.ds(0, CH)], buf, sem).wait()

    def scat(j, buf):
        pltpu.sync_copy(buf, acc.at[ceffv.at[j]], add=True)

    issue(0, buf0, sem0)
    issue(1, buf1, sem1)

    def pair(i, carry):
        wait(buf0, sem0)
        scat(2 * i, buf0)
        issue(2 * i + 2, buf0, sem0)
        wait(buf1, sem1)
        scat(2 * i + 1, buf1)
        issue(2 * i + 3, buf1, sem1)
        return carry

    lax.fori_loop(0, NCH2 // 2 - 1, pair, 0)
    wait(buf0, sem0)
    scat(NCH2 - 2, buf0)
    wait(buf1, sem1)
    scat(NCH2 - 1, buf1)
    plsc.subcore_barrier()
    pltpu.sync_copy(acc.at[pl.ds(s * ROWS_PER_TILE, ROWS_PER_TILE)],
                    out_hbm.at[pl.ds(s * ROWS_PER_TILE, ROWS_PER_TILE),
                               pl.ds(c * HD, HD)])


def _aggregate(s2, row_a, ceff_a, zerosd):
    return pl.kernel(
        _agg_body,
        out_type=jax.ShapeDtypeStruct((NPAD, D), jnp.float32),
        mesh=_sc_mesh(),
        scratch_types=[
            pltpu.VMEM((NCH2, CH), jnp.int32),
            pltpu.VMEM((NCH2, CH), jnp.int32),
            pltpu.VMEM((CH, HD), jnp.float32),
            pltpu.VMEM((CH, HD), jnp.float32),
            pltpu.VMEM_SHARED((NPAD, HD), jnp.float32),
            pltpu.VMEM_SHARED((NPAD, HD), jnp.float32),
            pltpu.SemaphoreType.DMA,
            pltpu.SemaphoreType.DMA,
        ],
        compiler_params=_SC_PARAMS,
    )(s2, row_a, ceff_a, zerosd)


def _fin_body(deg2_ref, dis_ref):
    full = deg2_ref[...]
    d = full[0:NPAD, 0:8] + full[NPAD:2 * NPAD, 0:8]
    r = lax.rsqrt(d)
    row = lax.broadcasted_iota(jnp.int32, (NPAD, 8), 0)
    dis_ref[...] = jnp.where(row < N, r, 0.0)


def _finalize_deg(deg2):
    return pl.pallas_call(
        _fin_body,
        out_shape=jax.ShapeDtypeStruct((NPAD, 8), jnp.float32),
    )(deg2)


def _dot(h, w):
    return lax.dot_general(h, w, (((1,), (1,)), ((), ())),
                           precision=lax.Precision.HIGHEST,
                           preferred_element_type=jnp.float32)


def _lin_body(dis_ref, h_ref, w_ref, b_ref, s_ref):
    dis = dis_ref[...][:, 0:1]
    s_ref[...] = dis * (_dot(h_ref[...], w_ref[...]) + b_ref[...])


def _linear(dis, h, w, b):
    return pl.pallas_call(
        _lin_body,
        grid=(GRID,),
        in_specs=[
            pl.BlockSpec((MB, 8), lambda i: (i, 0)),
            pl.BlockSpec((MB, D), lambda i: (i, 0)),
            pl.BlockSpec((D, D), lambda i: (0, 0)),
            pl.BlockSpec((1, D), lambda i: (0, 0)),
        ],
        out_specs=pl.BlockSpec((MB, D), lambda i: (i, 0)),
        out_shape=jax.ShapeDtypeStruct((NPAD, D), jnp.float32),
    )(dis, h, w, b)


def _resid_lin_body(h_ref, a_ref, dis_ref, w_ref, b_ref, hn_ref, s_ref):
    dis = dis_ref[...][:, 0:1]
    hn = h_ref[...] + jnp.maximum(dis * a_ref[...], 0.0)
    hn_ref[...] = hn
    s_ref[...] = dis * (_dot(hn, w_ref[...]) + b_ref[...])


def _resid_linear(h, acc, dis, w, b):
    return pl.pallas_call(
        _resid_lin_body,
        grid=(GRID,),
        in_specs=[
            pl.BlockSpec((MB, D), lambda i: (i, 0)),
            pl.BlockSpec((MB, D), lambda i: (i, 0)),
            pl.BlockSpec((MB, 8), lambda i: (i, 0)),
            pl.BlockSpec((D, D), lambda i: (0, 0)),
            pl.BlockSpec((1, D), lambda i: (0, 0)),
        ],
        out_specs=[
            pl.BlockSpec((MB, D), lambda i: (i, 0)),
            pl.BlockSpec((MB, D), lambda i: (i, 0)),
        ],
        out_shape=[
            jax.ShapeDtypeStruct((NPAD, D), jnp.float32),
            jax.ShapeDtypeStruct((NPAD, D), jnp.float32),
        ],
    )(h, acc, dis, w, b)


def _resid_body(h_ref, a_ref, dis_ref, hn_ref):
    dis = dis_ref[...][:, 0:1]
    hn_ref[...] = h_ref[...] + jnp.maximum(dis * a_ref[...], 0.0)


def _resid(h, acc, dis):
    return pl.pallas_call(
        _resid_body,
        grid=(GRID,),
        in_specs=[
            pl.BlockSpec((MB, D), lambda i: (i, 0)),
            pl.BlockSpec((MB, D), lambda i: (i, 0)),
            pl.BlockSpec((MB, 8), lambda i: (i, 0)),
        ],
        out_specs=pl.BlockSpec((MB, D), lambda i: (i, 0)),
        out_shape=jax.ShapeDtypeStruct((NPAD, D), jnp.float32),
    )(h, acc, dis)


def kernel(x, edge_index, W0, b0, W1, b1, W2, b2):
    x_p = jnp.pad(x, ((0, NPAD - N), (0, 0)))
    pad = jnp.zeros((EPAD - E,), jnp.int32)
    row_p = jnp.concatenate([edge_index[0], pad]).reshape(NW, NCHUNK, CH)
    col_p = jnp.concatenate([edge_index[1], pad]).reshape(NW, NCHUNK, CH)
    zeros16 = jnp.zeros((NPAD, 16), jnp.float32)
    zerosd = jnp.zeros((NPAD, HD), jnp.float32)

    ceff, deg2 = _preprocess(row_p, col_p, zeros16)
    dis = _finalize_deg(deg2)

    row_a = row_p.reshape(NS, NCH2, CH)
    ceff_a = ceff.reshape(NS, NCH2, CH)

    b0r = b0.reshape(1, D)
    b1r = b1.reshape(1, D)
    b2r = b2.reshape(1, D)

    s = _linear(dis, x_p, W0, b0r)
    acc = _aggregate(s, row_a, ceff_a, zerosd)
    h1, s = _resid_linear(x_p, acc, dis, W1, b1r)
    acc = _aggregate(s, row_a, ceff_a, zerosd)
    h2, s = _resid_linear(h1, acc, dis, W2, b2r)
    acc = _aggregate(s, row_a, ceff_a, zerosd)
    h3 = _resid(h2, acc, dis)
    return h3[:N]


# speedup vs baseline: 3.3108x; 1.0005x over previous
import jax
import jax.numpy as jnp
from jax import lax
from jax.experimental import pallas as pl
from jax.experimental.pallas import tpu as pltpu
from jax.experimental.pallas import tpu_sc as plsc

N = 10000
D = 128
HD = D // 2
E = 320000
NC = 2
NS = 16
NW = NC * NS
CH = 64
NCHUNK = 158
NCH2 = 2 * NCHUNK
EPAD = NW * NCHUNK * CH
NPAD = 10112
ROWS_PER_TILE = NPAD // NS
DUMMY = N
MB = 632
GRID = NPAD // MB


def _sc_mesh():
    return plsc.VectorSubcoreMesh(core_axis_name="c", subcore_axis_name="s",
                                  num_cores=NC, num_subcores=NS)


_SC_PARAMS = pltpu.CompilerParams(use_tc_tiling_on_sc=False)


def _pre_body(row_hbm, col_hbm, zeros16_hbm, ceff_hbm, deg2_hbm,
              rowv, colv, ceffv, reffv, onesv, degsh):
    c = lax.axis_index("c")
    s = lax.axis_index("s")
    tid = c * NS + s
    arow = tid // 2
    aoff = (tid % 2) * NCHUNK
    pltpu.sync_copy(row_hbm.at[arow, pl.ds(aoff, NCHUNK)], rowv)
    pltpu.sync_copy(col_hbm.at[arow, pl.ds(aoff, NCHUNK)], colv)
    pltpu.sync_copy(zeros16_hbm.at[pl.ds(s * ROWS_PER_TILE, ROWS_PER_TILE)],
                    degsh.at[pl.ds(s * ROWS_PER_TILE, ROWS_PER_TILE)])
    lanes = lax.iota(jnp.int32, 16)

    sub = CH // 16

    def cb(i, carry):
        j = i // sub
        k = (i % sub) * 16
        r = rowv[j, pl.ds(k, 16)]
        cc = colv[j, pl.ds(k, 16)]
        m = r != cc
        dv = DUMMY + lax.rem(i * 16 + lanes, NPAD - N)
        ceffv[j, pl.ds(k, 16)] = jnp.where(m, cc, dv)
        reffv[j, pl.ds(k, 16)] = jnp.where(m, r, dv)
        return carry

    lax.fori_loop(0, NCHUNK * sub, cb, 0)

    def ob(i, carry):
        onesv[i] = jnp.where(lanes == 0, 1.0, 0.0).astype(jnp.float32)
        return carry

    lax.fori_loop(0, CH, ob, 0)
    plsc.subcore_barrier()

    def sb(j, carry):
        pltpu.sync_copy(onesv, degsh.at[reffv.at[j]], add=True)
        return carry

    lax.fori_loop(0, NCHUNK, sb, 0)
    plsc.subcore_barrier()
    base = c * NPAD + s * ROWS_PER_TILE
    pltpu.sync_copy(degsh.at[pl.ds(s * ROWS_PER_TILE, ROWS_PER_TILE)],
                    deg2_hbm.at[pl.ds(base, ROWS_PER_TILE)])
    pltpu.sync_copy(ceffv, ceff_hbm.at[arow, pl.ds(aoff, NCHUNK)])


def _preprocess(row_p, col_p, zeros16):
    return pl.kernel(
        _pre_body,
        out_type=(
            jax.ShapeDtypeStruct((NS, NCH2, CH), jnp.int32),
            jax.ShapeDtypeStruct((NC * NPAD, 16), jnp.float32),
        ),
        mesh=_sc_mesh(),
        scratch_types=[
            pltpu.VMEM((NCHUNK, CH), jnp.int32),
            pltpu.VMEM((NCHUNK, CH), jnp.int32),
            pltpu.VMEM((NCHUNK, CH), jnp.int32),
            pltpu.VMEM((NCHUNK, CH), jnp.int32),
            pltpu.VMEM((CH, 16), jnp.float32),
            pltpu.VMEM_SHARED((NPAD, 16), jnp.float32),
        ],
        compiler_params=_SC_PARAMS,
    )(row_p, col_p, zeros16)


def _agg_body(s2_hbm, row_hbm, ceff_hbm, zerosd_hbm, out_hbm,
              rowv, ceffv, buf0, buf1,
              table, acc, sem0, sem1):
    c = lax.axis_index("c")
    s = lax.axis_index("s")
    pltpu.sync_copy(row_hbm.at[s], rowv)
    pltpu.sync_copy(ceff_hbm.at[s], ceffv)
    pltpu.sync_copy(s2_hbm.at[pl.ds(s * ROWS_PER_TILE, ROWS_PER_TILE),
                              pl.ds(c * HD, HD)],
                    table.at[pl.ds(s * ROWS_PER_TILE, ROWS_PER_TILE)])
    pltpu.sync_copy(zerosd_hbm.at[pl.ds(s * ROWS_PER_TILE, ROWS_PER_TILE)],
                    acc.at[pl.ds(s * ROWS_PER_TILE, ROWS_PER_TILE)])
    plsc.subcore_barrier()

    def issue(j, buf, sem):
        pltpu.async_copy(table.at[rowv.at[j]], buf, sem)

    def wait(buf, sem):
        pltpu.make_async_copy(table.at[pl.ds(0, CH)], buf, sem).wait()

    def scat(j, buf):
        pltpu.sync_copy(buf, acc.at[ceffv.at[j]], add=True)

    issue(0, buf0, sem0)
    issue(1, buf1, sem1)

    def pair(i, carry):
        wait(buf0, sem0)
        scat(2 * i, buf0)
        issue(2 * i + 2, buf0, sem0)
        wait(buf1, sem1)
        scat(2 * i + 1, buf1)
        issue(2 * i + 3, buf1, sem1)
        return carry

    lax.fori_loop(0, NCH2 // 2 - 1, pair, 0)
    wait(buf0, sem0)
    scat(NCH2 - 2, buf0)
    wait(buf1, sem1)
    scat(NCH2 - 1, buf1)
    plsc.subcore_barrier()
    pltpu.sync_copy(acc.at[pl.ds(s * ROWS_PER_TILE, ROWS_PER_TILE)],
                    out_hbm.at[pl.ds(s * ROWS_PER_TILE, ROWS_PER_TILE),
                               pl.ds(c * HD, HD)])


def _aggregate(s2, row_a, ceff_a, zerosd):
    return pl.kernel(
        _agg_body,
        out_type=jax.ShapeDtypeStruct((NPAD, D), jnp.float32),
        mesh=_sc_mesh(),
        scratch_types=[
            pltpu.VMEM((NCH2, CH), jnp.int32),
            pltpu.VMEM((NCH2, CH), jnp.int32),
            pltpu.VMEM((CH, HD), jnp.float32),
            pltpu.VMEM((CH, HD), jnp.float32),
            pltpu.VMEM_SHARED((NPAD, HD), jnp.float32),
            pltpu.VMEM_SHARED((NPAD, HD), jnp.float32),
            pltpu.SemaphoreType.DMA,
            pltpu.SemaphoreType.DMA,
        ],
        compiler_params=_SC_PARAMS,
    )(s2, row_a, ceff_a, zerosd)


def _fin_body(deg2_ref, dis_ref):
    full = deg2_ref[...]
    d = full[0:NPAD, 0:8] + full[NPAD:2 * NPAD, 0:8]
    r = lax.rsqrt(d)
    row = lax.broadcasted_iota(jnp.int32, (NPAD, 8), 0)
    dis_ref[...] = jnp.where(row < N, r, 0.0)


def _finalize_deg(deg2):
    return pl.pallas_call(
        _fin_body,
        out_shape=jax.ShapeDtypeStruct((NPAD, 8), jnp.float32),
    )(deg2)


def _dot(h, w):
    return lax.dot_general(h, w, (((1,), (1,)), ((), ())),
                           precision=lax.Precision.HIGHEST,
                           preferred_element_type=jnp.float32)


def _lin_body(dis_ref, h_ref, w_ref, b_ref, s_ref):
    dis = dis_ref[...][:, 0:1]
    s_ref[...] = dis * (_dot(h_ref[...], w_ref[...]) + b_ref[...])


def _linear(dis, h, w, b):
    return pl.pallas_call(
        _lin_body,
        grid=(GRID,),
        in_specs=[
            pl.BlockSpec((MB, 8), lambda i: (i, 0)),
            pl.BlockSpec((MB, D), lambda i: (i, 0)),
            pl.BlockSpec((D, D), lambda i: (0, 0)),
            pl.BlockSpec((1, D), lambda i: (0, 0)),
        ],
        out_specs=pl.BlockSpec((MB, D), lambda i: (i, 0)),
        out_shape=jax.ShapeDtypeStruct((NPAD, D), jnp.float32),
    )(dis, h, w, b)


def _resid_lin_body(h_ref, a_ref, dis_ref, w_ref, b_ref, hn_ref, s_ref):
    dis = dis_ref[...][:, 0:1]
    hn = h_ref[...] + jnp.maximum(dis * a_ref[...], 0.0)
    hn_ref[...] = hn
    s_ref[...] = dis * (_dot(hn, w_ref[...]) + b_ref[...])


def _resid_linear(h, acc, dis, w, b):
    return pl.pallas_call(
        _resid_lin_body,
        grid=(GRID,),
        in_specs=[
            pl.BlockSpec((MB, D), lambda i: (i, 0)),
            pl.BlockSpec((MB, D), lambda i: (i, 0)),
            pl.BlockSpec((MB, 8), lambda i: (i, 0)),
            pl.BlockSpec((D, D), lambda i: (0, 0)),
            pl.BlockSpec((1, D), lambda i: (0, 0)),
        ],
        out_specs=[
            pl.BlockSpec((MB, D), lambda i: (i, 0)),
            pl.BlockSpec((MB, D), lambda i: (i, 0)),
        ],
        out_shape=[
            jax.ShapeDtypeStruct((NPAD, D), jnp.float32),
            jax.ShapeDtypeStruct((NPAD, D), jnp.float32),
        ],
    )(h, acc, dis, w, b)


def _resid_body(h_ref, a_ref, dis_ref, hn_ref):
    dis = dis_ref[...][:, 0:1]
    hn_ref[...] = h_ref[...] + jnp.maximum(dis * a_ref[...], 0.0)


def _resid(h, acc, dis):
    return pl.pallas_call(
        _resid_body,
        grid=(GRID,),
        in_specs=[
            pl.BlockSpec((MB, D), lambda i: (i, 0)),
            pl.BlockSpec((MB, D), lambda i: (i, 0)),
            pl.BlockSpec((MB, 8), lambda i: (i, 0)),
        ],
        out_specs=pl.BlockSpec((MB, D), lambda i: (i, 0)),
        out_shape=jax.ShapeDtypeStruct((NPAD, D), jnp.float32),
    )(h, acc, dis)


def kernel(x, edge_index, W0, b0, W1, b1, W2, b2):
    x_p = jnp.pad(x, ((0, NPAD - N), (0, 0)))
    pad = jnp.zeros((EPAD - E,), jnp.int32)
    row_p = jnp.concatenate([edge_index[0], pad]).reshape(NS, NCH2, CH)
    col_p = jnp.concatenate([edge_index[1], pad]).reshape(NS, NCH2, CH)
    zeros16 = jnp.zeros((NPAD, 16), jnp.float32)
    zerosd = jnp.zeros((NPAD, HD), jnp.float32)

    ceff, deg2 = _preprocess(row_p, col_p, zeros16)
    dis = _finalize_deg(deg2)
    row_a = row_p
    ceff_a = ceff

    b0r = b0.reshape(1, D)
    b1r = b1.reshape(1, D)
    b2r = b2.reshape(1, D)

    s = _linear(dis, x_p, W0, b0r)
    acc = _aggregate(s, row_a, ceff_a, zerosd)
    h1, s = _resid_linear(x_p, acc, dis, W1, b1r)
    acc = _aggregate(s, row_a, ceff_a, zerosd)
    h2, s = _resid_linear(h1, acc, dis, W2, b2r)
    acc = _aggregate(s, row_a, ceff_a, zerosd)
    h3 = _resid(h2, acc, dis)
    return h3[:N]
